# baseline (speedup 1.0000x reference)
import functools

import numpy as np
import jax
import jax.numpy as jnp
from jax import lax
from jax.experimental import pallas as pl
from jax.experimental.pallas import tpu as pltpu
from jax.experimental.pallas import tpu_sc as plsc

_VOCAB = 100000
_M = 64
_B = 1024
_C = 200
_MAT_FACTOR = 8.0
_POS_FACTOR = 1.0

_NBH = 4
_NCQ = 8
_BB = _B // _NBH
_CCG = _C // _NCQ

_denom = 10000.0 ** np.linspace(0.0, 1.0, _M)
_arg = np.arange(_C, dtype=np.float64)[:, None] / _denom[None, :]
_PTAB = np.concatenate(
    [np.sin(_arg), np.cos(_arg)], axis=0
).astype(np.float32) * np.float32(_POS_FACTOR)


def _body(emb_hbm, seq_hbm, tok_hbm, ptab_hbm, out_hbm,
          idx_t, pidx_t, erow0, prow0, erow1, prow1, slab,
          isem, gsem0, gsem1, osem):
    wid = lax.axis_index("s") * 2 + lax.axis_index("c")
    bh = wid >> 3
    cq = wid & 7
    b0 = bh * _BB
    c0 = cq * _CCG

    erow = (erow0, erow1)
    prow = (prow0, prow1)
    gsem = (gsem0, gsem1)

    cp1 = pltpu.async_copy(
        seq_hbm.at[pl.ds(c0, _CCG), pl.ds(b0, _BB)], idx_t, isem)
    cp2 = pltpu.async_copy(
        tok_hbm.at[pl.ds(c0, _CCG), pl.ds(b0, _BB)], pidx_t, isem)
    cp1.wait()
    cp2.wait()

    zero16 = lax.iota(jnp.int32, 16) & 0

    @plsc.parallel_loop(0, _CCG * (_BB // 16), unroll=4)
    def _mk_pidx(j):
        cl = j >> 4
        g = j & 15
        pat = zero16 + ((c0 + cl) & 1) * _C
        sl = pl.ds(g * 16, 16)
        pidx_t[cl, sl] = pidx_t[cl, sl] + pat

    def fire_gathers(cl, s):
        cps = []
        for h in range(_BB // 128):
            sl = pl.ds(h * 128, 128)
            cps.append(pltpu.async_copy(
                emb_hbm.at[idx_t.at[cl, sl]], erow[s].at[sl], gsem[s]))
            cps.append(pltpu.async_copy(
                ptab_hbm.at[pidx_t.at[cl, sl]], prow[s].at[sl], gsem[s]))
        return cps

    iota = lax.iota(jnp.int32, 16)

    def fuse(s):
        @plsc.parallel_loop(0, _BB * (_M // 16), unroll=4)
        def _f(vid):
            bb = vid >> 2
            m0 = (vid & 3) * 16
            val = erow[s][bb, pl.ds(m0, 16)] * _MAT_FACTOR \
                + prow[s][bb, pl.ds(m0, 16)]
            mv = m0 + iota
            mtv = mv >> 3
            rv = mv & 7
            bt2v = zero16 + (bb >> 7)
            lv = zero16 + (bb & 127)
            plsc.store_scatter(slab, [mtv, bt2v, rv, lv], val)

    pending = fire_gathers(0, 0)
    pstore = None
    for cl in range(_CCG):
        s = cl & 1
        nxt = fire_gathers(cl + 1, 1 - s) if cl + 1 < _CCG else None
        for cp in pending:
            cp.wait()
        if pstore is not None:
            pstore.wait()
        fuse(s)
        pstore = pltpu.async_copy(
            slab, out_hbm.at[c0 + cl, :, pl.ds(bh * 2, 2)], osem)
        pending = nxt
    pstore.wait()


@jax.jit
def _run(emb, seq, tok, ptab):
    mesh = plsc.VectorSubcoreMesh(core_axis_name="c", subcore_axis_name="s")
    f = functools.partial(
        pl.kernel,
        out_type=jax.ShapeDtypeStruct((_C, 8, 8, 8, 128), jnp.float32),
        mesh=mesh,
        scratch_types=[
            pltpu.VMEM((_CCG, _BB), jnp.int32),
            pltpu.VMEM((_CCG, _BB), jnp.int32),
            pltpu.VMEM((_BB, _M), jnp.float32),
            pltpu.VMEM((_BB, _M), jnp.float32),
            pltpu.VMEM((_BB, _M), jnp.float32),
            pltpu.VMEM((_BB, _M), jnp.float32),
            pltpu.VMEM((8, 2, 8, 128), jnp.float32),
            pltpu.SemaphoreType.DMA,
            pltpu.SemaphoreType.DMA,
            pltpu.SemaphoreType.DMA,
            pltpu.SemaphoreType.DMA,
        ],
        compiler_params=pltpu.CompilerParams(
            use_tc_tiling_on_sc=False, needs_layout_passes=False),
    )(_body)
    return f(emb, seq, tok, ptab)


def kernel(embed_mat, seq_tokens, tokids):
    seq_t = seq_tokens.astype(jnp.int32).T
    tok_t = tokids.astype(jnp.int32).T
    ptab = jnp.asarray(_PTAB)
    outp = _run(embed_mat, seq_t, tok_t, ptab)
    return outp.transpose(2, 4, 0, 1, 3).reshape(_B, _C, _M)


# speedup vs baseline: 1.0695x; 1.0695x over previous
import functools

import numpy as np
import jax
import jax.numpy as jnp
from jax import lax
from jax.experimental import pallas as pl
from jax.experimental.pallas import tpu as pltpu
from jax.experimental.pallas import tpu_sc as plsc

_VOCAB = 100000
_M = 64
_B = 1024
_C = 200
_MAT_FACTOR = 8.0
_POS_FACTOR = 1.0

_NBH = 4
_NCQ = 8
_BB = _B // _NBH
_CCG = _C // _NCQ

_denom = 10000.0 ** np.linspace(0.0, 1.0, _M)
_arg = np.arange(_C, dtype=np.float64)[:, None] / _denom[None, :]
_PTAB = np.concatenate(
    [np.sin(_arg), np.cos(_arg)], axis=0
).astype(np.float32) * np.float32(_POS_FACTOR)


def _body(emb_hbm, seq_hbm, tok_hbm, ptab_hbm, out_hbm,
          idx_t, pidx_t, erow0, prow0, erow1, prow1, slab,
          isem, gsem0, gsem1, osem):
    wid = lax.axis_index("s") * 2 + lax.axis_index("c")
    bh = wid >> 3
    cq = wid & 7
    b0 = bh * _BB
    c0 = cq * _CCG

    erow = (erow0, erow1)
    prow = (prow0, prow1)
    gsem = (gsem0, gsem1)

    cp1 = pltpu.async_copy(
        seq_hbm.at[pl.ds(c0, _CCG), pl.ds(b0, _BB)], idx_t, isem)
    cp2 = pltpu.async_copy(
        tok_hbm.at[pl.ds(c0, _CCG), pl.ds(b0, _BB)], pidx_t, isem)
    cp1.wait()
    cp2.wait()

    zero16 = lax.iota(jnp.int32, 16) & 0

    @plsc.parallel_loop(0, _CCG * (_BB // 16), unroll=4)
    def _mk_pidx(j):
        cl = j >> 4
        g = j & 15
        pat = zero16 + ((c0 + cl) & 1) * _C
        sl = pl.ds(g * 16, 16)
        pidx_t[cl, sl] = pidx_t[cl, sl] + pat

    def fire_gathers(cl, s):
        cps = []
        for h in range(_BB // 128):
            sl = pl.ds(h * 128, 128)
            cps.append(pltpu.async_copy(
                emb_hbm.at[idx_t.at[cl, sl]], erow[s].at[sl], gsem[s]))
            cps.append(pltpu.async_copy(
                ptab_hbm.at[pidx_t.at[cl, sl]], prow[s].at[sl], gsem[s]))
        return cps

    iota = lax.iota(jnp.int32, 16)

    def fuse(s):
        for ms in range(_M // 16):
            mtv = (ms * 16 + iota) >> 3
            rv = (ms * 16 + iota) & 7

            @plsc.parallel_loop(0, _BB, unroll=8)
            def _f(bb, _mtv=mtv, _rv=rv, _ms=ms):
                sl = pl.ds(_ms * 16, 16)
                val = erow[s][bb, sl] * _MAT_FACTOR + prow[s][bb, sl]
                bt2v = zero16 + (bb >> 7)
                lv = zero16 + (bb & 127)
                plsc.store_scatter(slab, [_mtv, bt2v, _rv, lv], val)

    pending = fire_gathers(0, 0)
    pstore = None
    for cl in range(_CCG):
        s = cl & 1
        nxt = fire_gathers(cl + 1, 1 - s) if cl + 1 < _CCG else None
        for cp in pending:
            cp.wait()
        if pstore is not None:
            pstore.wait()
        fuse(s)
        pstore = pltpu.async_copy(
            slab, out_hbm.at[c0 + cl, :, pl.ds(bh * 2, 2)], osem)
        pending = nxt
    pstore.wait()


@jax.jit
def _run(emb, seq, tok, ptab):
    mesh = plsc.VectorSubcoreMesh(core_axis_name="c", subcore_axis_name="s")
    f = functools.partial(
        pl.kernel,
        out_type=jax.ShapeDtypeStruct((_C, 8, 8, 8, 128), jnp.float32),
        mesh=mesh,
        scratch_types=[
            pltpu.VMEM((_CCG, _BB), jnp.int32),
            pltpu.VMEM((_CCG, _BB), jnp.int32),
            pltpu.VMEM((_BB, _M), jnp.float32),
            pltpu.VMEM((_BB, _M), jnp.float32),
            pltpu.VMEM((_BB, _M), jnp.float32),
            pltpu.VMEM((_BB, _M), jnp.float32),
            pltpu.VMEM((8, 2, 8, 128), jnp.float32),
            pltpu.SemaphoreType.DMA,
            pltpu.SemaphoreType.DMA,
            pltpu.SemaphoreType.DMA,
            pltpu.SemaphoreType.DMA,
        ],
        compiler_params=pltpu.CompilerParams(
            use_tc_tiling_on_sc=False, needs_layout_passes=False),
    )(_body)
    return f(emb, seq, tok, ptab)


def kernel(embed_mat, seq_tokens, tokids):
    seq_t = seq_tokens.astype(jnp.int32).T
    tok_t = tokids.astype(jnp.int32).T
    ptab = jnp.asarray(_PTAB)
    outp = _run(embed_mat, seq_t, tok_t, ptab)
    return outp.transpose(2, 4, 0, 1, 3).reshape(_B, _C, _M)


# speedup vs baseline: 1.1254x; 1.0522x over previous
import functools

import numpy as np
import jax
import jax.numpy as jnp
from jax import lax
from jax.experimental import pallas as pl
from jax.experimental.pallas import tpu as pltpu
from jax.experimental.pallas import tpu_sc as plsc

_VOCAB = 100000
_M = 64
_B = 1024
_C = 200
_MAT_FACTOR = 8.0
_POS_FACTOR = 1.0

_NBH = 4
_NCQ = 8
_BB = _B // _NBH
_CCG = _C // _NCQ

_denom = 10000.0 ** np.linspace(0.0, 1.0, _M)
_arg = np.arange(_C, dtype=np.float64)[:, None] / _denom[None, :]
_PTAB = np.concatenate(
    [np.sin(_arg), np.cos(_arg)], axis=0
).astype(np.float32) * np.float32(_POS_FACTOR)


def _body(emb_hbm, seq_hbm, tok_hbm, ptab_hbm, out_hbm,
          idx_t, pidx_t, erow0, prow0, erow1, prow1, slab0, slab1,
          isem, gsem0, gsem1, osem0, osem1):
    wid = lax.axis_index("s") * 2 + lax.axis_index("c")
    bh = wid >> 3
    cq = wid & 7
    b0 = bh * _BB
    c0 = cq * _CCG

    erow = (erow0, erow1)
    prow = (prow0, prow1)
    gsem = (gsem0, gsem1)

    cp1 = pltpu.async_copy(
        seq_hbm.at[pl.ds(c0, _CCG), pl.ds(b0, _BB)], idx_t, isem)
    cp2 = pltpu.async_copy(
        tok_hbm.at[pl.ds(c0, _CCG), pl.ds(b0, _BB)], pidx_t, isem)
    cp1.wait()
    cp2.wait()

    zero16 = lax.iota(jnp.int32, 16) & 0

    @plsc.parallel_loop(0, _CCG * (_BB // 16), unroll=4)
    def _mk_pidx(j):
        cl = j >> 4
        g = j & 15
        pat = zero16 + ((c0 + cl) & 1) * _C
        sl = pl.ds(g * 16, 16)
        pidx_t[cl, sl] = pidx_t[cl, sl] + pat

    def fire_gathers(cl, s):
        cps = []
        for h in range(_BB // 128):
            sl = pl.ds(h * 128, 128)
            cps.append(pltpu.async_copy(
                emb_hbm.at[idx_t.at[cl, sl]], erow[s].at[sl], gsem[s]))
            cps.append(pltpu.async_copy(
                ptab_hbm.at[pidx_t.at[cl, sl]], prow[s].at[sl], gsem[s]))
        return cps

    iota = lax.iota(jnp.int32, 16)

    slab = (slab0, slab1)
    osem = (osem0, osem1)

    def fuse(s):
        for ms in range(_M // 16):
            mtv = (ms * 16 + iota) >> 3
            rv = (ms * 16 + iota) & 7

            @plsc.parallel_loop(0, _BB, unroll=8)
            def _f(bb, _mtv=mtv, _rv=rv, _ms=ms):
                sl = pl.ds(_ms * 16, 16)
                val = erow[s][bb, sl] * _MAT_FACTOR + prow[s][bb, sl]
                bt2v = zero16 + (bb >> 7)
                lv = zero16 + (bb & 127)
                plsc.store_scatter(slab[s], [_mtv, bt2v, _rv, lv], val)

    pending = fire_gathers(0, 0)
    pstore = [None, None]
    for cl in range(_CCG):
        s = cl & 1
        nxt = fire_gathers(cl + 1, 1 - s) if cl + 1 < _CCG else None
        for cp in pending:
            cp.wait()
        if pstore[s] is not None:
            pstore[s].wait()
        fuse(s)
        pstore[s] = pltpu.async_copy(
            slab[s], out_hbm.at[c0 + cl, :, pl.ds(bh * 2, 2)], osem[s])
        pending = nxt
    for cp in pstore:
        if cp is not None:
            cp.wait()


@jax.jit
def _run(emb, seq, tok, ptab):
    mesh = plsc.VectorSubcoreMesh(core_axis_name="c", subcore_axis_name="s")
    f = functools.partial(
        pl.kernel,
        out_type=jax.ShapeDtypeStruct((_C, 8, 8, 8, 128), jnp.float32),
        mesh=mesh,
        scratch_types=[
            pltpu.VMEM((_CCG, _BB), jnp.int32),
            pltpu.VMEM((_CCG, _BB), jnp.int32),
            pltpu.VMEM((_BB, _M), jnp.float32),
            pltpu.VMEM((_BB, _M), jnp.float32),
            pltpu.VMEM((_BB, _M), jnp.float32),
            pltpu.VMEM((_BB, _M), jnp.float32),
            pltpu.VMEM((8, 2, 8, 128), jnp.float32),
            pltpu.VMEM((8, 2, 8, 128), jnp.float32),
            pltpu.SemaphoreType.DMA,
            pltpu.SemaphoreType.DMA,
            pltpu.SemaphoreType.DMA,
            pltpu.SemaphoreType.DMA,
            pltpu.SemaphoreType.DMA,
        ],
        compiler_params=pltpu.CompilerParams(
            use_tc_tiling_on_sc=False, needs_layout_passes=False),
    )(_body)
    return f(emb, seq, tok, ptab)


def kernel(embed_mat, seq_tokens, tokids):
    seq_t = seq_tokens.astype(jnp.int32).T
    tok_t = tokids.astype(jnp.int32).T
    ptab = jnp.asarray(_PTAB)
    outp = _run(embed_mat, seq_t, tok_t, ptab)
    return outp.transpose(2, 4, 0, 1, 3).reshape(_B, _C, _M)
